# baseline (speedup 1.0000x reference)
import functools
import math

import jax
import jax.numpy as jnp
from jax import lax
from jax.experimental import pallas as pl
from jax.experimental.pallas import tpu as pltpu
from jax.experimental.pallas import tpu_sc as plsc

NUM_ITEM = 100000
D = 128
B = 1024
WIN = 50

NC = 2
NS = 16
NW = NC * NS
BPW = B // NW
LANES = 16

TN = 2000
NT = NUM_ITEM // TN
LOG2E = math.log2(math.e)

_LOG_S2 = math.log(NUM_ITEM + 1.0 + (math.e - 2.0) / 2.0)


def _sc_pool_body(seq_idx_hbm, item_idx_hbm, tgt_idx_hbm,
                  w_seq_hbm, w_item_hbm, w_proj_hbm, b16_hbm,
                  c_hbm, wpt_hbm, bt_hbm,
                  seq_idx_v, item_idx_v, tgt_idx_v,
                  seq_rows_v, item_rows_v, wpt_rows_v, c_v, bt_v,
                  sem):
    wid = lax.axis_index("s") * NC + lax.axis_index("c")
    base = wid * BPW

    pltpu.sync_copy(seq_idx_hbm.at[pl.ds(base, BPW)], seq_idx_v)
    pltpu.sync_copy(item_idx_hbm.at[pl.ds(base, BPW)], item_idx_v)
    pltpu.sync_copy(tgt_idx_hbm.at[pl.ds(base, BPW)], tgt_idx_v)

    cp1 = pltpu.async_copy(w_seq_hbm.at[seq_idx_v], seq_rows_v, sem)
    cp2 = pltpu.async_copy(w_proj_hbm.at[tgt_idx_v], wpt_rows_v, sem)
    cp3 = pltpu.async_copy(b16_hbm.at[tgt_idx_v], bt_v, sem)
    cp1.wait()
    cp2.wait()
    cp3.wait()

    scale = LOG2E / (WIN + 1.0)
    for p in range(3):
        pltpu.async_copy(w_item_hbm.at[item_idx_v.at[p]],
                         item_rows_v.at[p], sem)

    def per_row(i, carry):
        par = lax.rem(i, 4)
        buf = item_rows_v.at[par]
        pltpu.make_async_copy(w_item_hbm.at[item_idx_v.at[i]], buf,
                              sem).wait()

        @pl.when(i + 3 < BPW)
        def _prefetch():
            pltpu.async_copy(w_item_hbm.at[item_idx_v.at[i + 3]],
                             item_rows_v.at[lax.rem(i + 3, 4)], sem)

        nch = D // LANES
        sls = [pl.ds(ch * LANES, LANES) for ch in range(nch)]

        def add_rows(g, accs):
            r = g * 10
            for k in range(10):
                accs = tuple(accs[ch] + buf[r + k, sls[ch]]
                             for ch in range(nch))
            return accs

        accs = lax.fori_loop(
            0, WIN // 10, add_rows,
            tuple(seq_rows_v[i, sls[ch]] for ch in range(nch)))
        for ch in range(nch):
            c_v[i, sls[ch]] = accs[ch] * scale
        return carry

    lax.fori_loop(0, BPW, per_row, 0)

    pltpu.sync_copy(c_v, c_hbm.at[pl.ds(base, BPW)])
    pltpu.sync_copy(wpt_rows_v, wpt_hbm.at[pl.ds(base, BPW)])
    pltpu.sync_copy(bt_v, bt_hbm.at[pl.ds(base, BPW)])


def _sc_pool(seq_idx, item_idx, tgt_idx, w_seq, w_item, w_proj, b16):
    mesh = plsc.VectorSubcoreMesh(core_axis_name="c", subcore_axis_name="s")
    f = pl.kernel(
        _sc_pool_body,
        out_type=[
            jax.ShapeDtypeStruct((B, D), jnp.float32),
            jax.ShapeDtypeStruct((B, D), jnp.float32),
            jax.ShapeDtypeStruct((B,), jnp.float32),
        ],
        mesh=mesh,
        scratch_types=[
            pltpu.VMEM((BPW,), jnp.int32),
            pltpu.VMEM((BPW, WIN), jnp.int32),
            pltpu.VMEM((BPW,), jnp.int32),
            pltpu.VMEM((BPW, D), jnp.float32),
            pltpu.VMEM((4, WIN, D), jnp.float32),
            pltpu.VMEM((BPW, D), jnp.float32),
            pltpu.VMEM((BPW, D), jnp.float32),
            pltpu.VMEM((BPW,), jnp.float32),
            pltpu.SemaphoreType.DMA,
        ],
    )
    return f(seq_idx, item_idx, tgt_idx, w_seq, w_item, w_proj, b16)


def _tc_proj_body(c_ref, w_ref, b_ref, wpt_ref, bt_ref, loss_ref,
                  m_ref, z_ref, da_ref, db_ref):
    j = pl.program_id(0)

    @pl.when(j == 0)
    def _init():
        m_ref[...] = jnp.full((B, 1), -jnp.inf, jnp.float32)
        z_ref[...] = jnp.zeros((B, 1), jnp.float32)

    even = lax.rem(j, 2) == 0
    cb = c_ref[...].astype(jnp.bfloat16)
    dn = (((1,), (1,)), ((), ()))

    @pl.when(even & (j < NT))
    def _dot_a():
        da_ref[...] = lax.dot_general(
            cb, w_ref[...].astype(jnp.bfloat16), dn,
            preferred_element_type=jnp.float32)

    @pl.when(jnp.logical_not(even) & (j < NT))
    def _dot_b():
        db_ref[...] = lax.dot_general(
            cb, w_ref[...].astype(jnp.bfloat16), dn,
            preferred_element_type=jnp.float32)

    def _update(dref):
        b = b_ref[0]
        d = dref[...]
        m_old = m_ref[...]
        s_u = jnp.sum(jnp.exp2(d + b), axis=1, keepdims=True)
        s = s_u * jnp.exp2(-m_old)
        trip = (jnp.any(s > 2.0 ** 100) | jnp.any(s_u < 2.0 ** -100)
                | jnp.any(jnp.isnan(s)))

        @pl.when(jnp.logical_not(trip))
        def _fast():
            z_ref[...] = z_ref[...] + s

        @pl.when(trip)
        def _rebase():
            bound = jnp.max(d, axis=1, keepdims=True) + jnp.max(b)
            m_new = jnp.maximum(m_old, bound)
            z_ref[...] = (z_ref[...] * jnp.exp2(m_old - m_new)
                          + jnp.sum(jnp.exp2((d + b) - m_new),
                                    axis=1, keepdims=True))
            m_ref[...] = m_new

    @pl.when(even & (j > 0))
    def _upd_b():
        _update(db_ref)

    @pl.when(jnp.logical_not(even))
    def _upd_a():
        _update(da_ref)

    @pl.when(j == NT)
    def _fin():
        vt = jnp.sum(c_ref[...] * wpt_ref[...], axis=1, keepdims=True)
        vt = vt + bt_ref[...] * LOG2E
        out_t = jnp.exp2(vt - m_ref[...]) / z_ref[...]
        loss_ref[...] = jnp.reshape(
            _LOG_S2 - jnp.sum(out_t) * (1.0 / B), (1, 1))


def _tc_proj(c, w_proj, b2d, wpt, bt2d):
    return pl.pallas_call(
        _tc_proj_body,
        grid=(NT + 1,),
        in_specs=[
            pl.BlockSpec((B, D), lambda j: (0, 0)),
            pl.BlockSpec((TN, D), lambda j: (jnp.minimum(j, NT - 1), 0)),
            pl.BlockSpec((1, 1, TN),
                         lambda j: (jnp.maximum(j - 1, 0), 0, 0)),
            pl.BlockSpec((B, D), lambda j: (0, 0)),
            pl.BlockSpec((B, 1), lambda j: (0, 0)),
        ],
        out_specs=pl.BlockSpec((1, 1), lambda j: (0, 0)),
        out_shape=jax.ShapeDtypeStruct((1, 1), jnp.float32),
        scratch_shapes=[
            pltpu.VMEM((B, 1), jnp.float32),
            pltpu.VMEM((B, 1), jnp.float32),
            pltpu.VMEM((B, TN), jnp.float32),
            pltpu.VMEM((B, TN), jnp.float32),
        ],
        compiler_params=pltpu.CompilerParams(
            dimension_semantics=("arbitrary",)),
    )(c, w_proj, b2d, wpt, bt2d)


def kernel(seq_index, item_indicies, target_index, W_seq, W_item, W_proj,
           b_proj):
    seq_index = seq_index.astype(jnp.int32)
    item_indicies = item_indicies.astype(jnp.int32)
    target_index = target_index.astype(jnp.int32)
    c, wpt, bt = _sc_pool(seq_index, item_indicies, target_index,
                          W_seq, W_item, W_proj, b_proj)
    loss = _tc_proj(c, W_proj,
                    (b_proj * LOG2E).reshape(NT, 1, TN), wpt,
                    bt.reshape(B, 1))
    return loss[0, 0]


# speedup vs baseline: 1.0234x; 1.0234x over previous
import functools
import math

import jax
import jax.numpy as jnp
from jax import lax
from jax.experimental import pallas as pl
from jax.experimental.pallas import tpu as pltpu
from jax.experimental.pallas import tpu_sc as plsc

NUM_ITEM = 100000
D = 128
B = 1024
WIN = 50

NC = 2
NS = 16
NW = NC * NS
BPW = B // NW
LANES = 16

TN = 2000
NT = NUM_ITEM // TN
LOG2E = math.log2(math.e)

_LOG_S2 = math.log(NUM_ITEM + 1.0 + (math.e - 2.0) / 2.0)


def _sc_pool_body(seq_idx_hbm, item_idx_hbm, tgt_idx_hbm,
                  w_seq_hbm, w_item_hbm, w_proj_hbm, b16_hbm,
                  c_hbm, wpt_hbm, bt_hbm,
                  seq_idx_v, item_idx_v, tgt_idx_v,
                  seq_rows_v, item_rows_v, wpt_rows_v, c_v, bt_v,
                  sem):
    wid = lax.axis_index("s") * NC + lax.axis_index("c")
    base = wid * BPW

    pltpu.sync_copy(seq_idx_hbm.at[pl.ds(base, BPW)], seq_idx_v)
    pltpu.sync_copy(item_idx_hbm.at[pl.ds(base, BPW)], item_idx_v)
    pltpu.sync_copy(tgt_idx_hbm.at[pl.ds(base, BPW)], tgt_idx_v)

    cp1 = pltpu.async_copy(w_seq_hbm.at[seq_idx_v], seq_rows_v, sem)
    cp2 = pltpu.async_copy(w_proj_hbm.at[tgt_idx_v], wpt_rows_v, sem)
    cp3 = pltpu.async_copy(b16_hbm.at[tgt_idx_v], bt_v, sem)
    cp1.wait()
    cp2.wait()
    cp3.wait()

    scale = LOG2E / (WIN + 1.0)
    for p in range(3):
        pltpu.async_copy(w_item_hbm.at[item_idx_v.at[p]],
                         item_rows_v.at[p], sem)

    def per_row(i, carry):
        par = lax.rem(i, 4)
        buf = item_rows_v.at[par]
        pltpu.make_async_copy(w_item_hbm.at[item_idx_v.at[i]], buf,
                              sem).wait()

        @pl.when(i + 3 < BPW)
        def _prefetch():
            pltpu.async_copy(w_item_hbm.at[item_idx_v.at[i + 3]],
                             item_rows_v.at[lax.rem(i + 3, 4)], sem)

        nch = D // LANES
        sls = [pl.ds(ch * LANES, LANES) for ch in range(nch)]

        def add_rows(g, accs):
            r = g * 10
            for k in range(10):
                accs = tuple(accs[ch] + buf[r + k, sls[ch]]
                             for ch in range(nch))
            return accs

        accs = lax.fori_loop(
            0, WIN // 10, add_rows,
            tuple(seq_rows_v[i, sls[ch]] for ch in range(nch)))
        for ch in range(nch):
            c_v[i, sls[ch]] = accs[ch] * scale
        return carry

    lax.fori_loop(0, BPW, per_row, 0)

    pltpu.sync_copy(c_v, c_hbm.at[pl.ds(base, BPW)])
    pltpu.sync_copy(wpt_rows_v, wpt_hbm.at[pl.ds(base, BPW)])
    pltpu.sync_copy(bt_v, bt_hbm.at[pl.ds(base, BPW)])


def _sc_pool(seq_idx, item_idx, tgt_idx, w_seq, w_item, w_proj, b16):
    mesh = plsc.VectorSubcoreMesh(core_axis_name="c", subcore_axis_name="s")
    f = pl.kernel(
        _sc_pool_body,
        out_type=[
            jax.ShapeDtypeStruct((B, D), jnp.float32),
            jax.ShapeDtypeStruct((B, D), jnp.float32),
            jax.ShapeDtypeStruct((B,), jnp.float32),
        ],
        mesh=mesh,
        scratch_types=[
            pltpu.VMEM((BPW,), jnp.int32),
            pltpu.VMEM((BPW, WIN), jnp.int32),
            pltpu.VMEM((BPW,), jnp.int32),
            pltpu.VMEM((BPW, D), jnp.float32),
            pltpu.VMEM((4, WIN, D), jnp.float32),
            pltpu.VMEM((BPW, D), jnp.float32),
            pltpu.VMEM((BPW, D), jnp.float32),
            pltpu.VMEM((BPW,), jnp.float32),
            pltpu.SemaphoreType.DMA,
        ],
    )
    return f(seq_idx, item_idx, tgt_idx, w_seq, w_item, w_proj, b16)


NP = NT // 2


def _tc_proj_body(c_ref, w_ref, ba_ref, bb_ref, wpt_ref, bt_ref, loss_ref,
                  m_ref, z_ref, db_ref):
    i = pl.program_id(0)

    @pl.when(i == 0)
    def _init():
        m_ref[...] = jnp.full((B, 1), -jnp.inf, jnp.float32)
        z_ref[...] = jnp.zeros((B, 1), jnp.float32)

    cb = c_ref[...].astype(jnp.bfloat16)
    dn = (((1,), (1,)), ((), ()))
    ba = ba_ref[0]
    bb = bb_ref[0]
    d_a = lax.dot_general(cb, w_ref[pl.ds(0, TN), :].astype(jnp.bfloat16),
                          dn, preferred_element_type=jnp.float32)
    su_b = jnp.sum(jnp.exp2(db_ref[...] + bb), axis=1, keepdims=True)
    d_b = lax.dot_general(cb, w_ref[pl.ds(TN, TN), :].astype(jnp.bfloat16),
                          dn, preferred_element_type=jnp.float32)
    su_a = jnp.sum(jnp.exp2(d_a + ba), axis=1, keepdims=True)

    @pl.when(i > 0)
    def _upd_b():
        m_old = m_ref[...]
        s = su_b * jnp.exp2(-m_old)
        trip = (jnp.any(s > 2.0 ** 100) | jnp.any(su_b < 2.0 ** -100)
                | jnp.any(jnp.isnan(s)))

        @pl.when(jnp.logical_not(trip))
        def _fast():
            z_ref[...] = z_ref[...] + s

        @pl.when(trip)
        def _rebase():
            d = db_ref[...]
            bound = jnp.max(d, axis=1, keepdims=True) + jnp.max(bb)
            m_new = jnp.maximum(m_old, bound)
            z_ref[...] = (z_ref[...] * jnp.exp2(m_old - m_new)
                          + jnp.sum(jnp.exp2((d + bb) - m_new),
                                    axis=1, keepdims=True))
            m_ref[...] = m_new

    @pl.when(i < NP)
    def _upd_a():
        m_old = m_ref[...]
        s = su_a * jnp.exp2(-m_old)
        trip = (jnp.any(s > 2.0 ** 100) | jnp.any(su_a < 2.0 ** -100)
                | jnp.any(jnp.isnan(s)))

        @pl.when(jnp.logical_not(trip))
        def _fast():
            z_ref[...] = z_ref[...] + s

        @pl.when(trip)
        def _rebase():
            bound = jnp.max(d_a, axis=1, keepdims=True) + jnp.max(ba)
            m_new = jnp.maximum(m_old, bound)
            z_ref[...] = (z_ref[...] * jnp.exp2(m_old - m_new)
                          + jnp.sum(jnp.exp2((d_a + ba) - m_new),
                                    axis=1, keepdims=True))
            m_ref[...] = m_new

    @pl.when(i < NP)
    def _store_b():
        db_ref[...] = d_b

    @pl.when(i == NP)
    def _fin():
        vt = jnp.sum(c_ref[...] * wpt_ref[...], axis=1, keepdims=True)
        vt = vt + bt_ref[...] * LOG2E
        out_t = jnp.exp2(vt - m_ref[...]) / z_ref[...]
        loss_ref[...] = jnp.reshape(
            _LOG_S2 - jnp.sum(out_t) * (1.0 / B), (1, 1))


def _tc_proj(c, w_proj, b2d, wpt, bt2d):
    return pl.pallas_call(
        _tc_proj_body,
        grid=(NP + 1,),
        in_specs=[
            pl.BlockSpec((B, D), lambda i: (0, 0)),
            pl.BlockSpec((2 * TN, D), lambda i: (jnp.minimum(i, NP - 1), 0)),
            pl.BlockSpec((1, 1, TN),
                         lambda i: (jnp.minimum(2 * i, NT - 1), 0, 0)),
            pl.BlockSpec((1, 1, TN),
                         lambda i: (jnp.maximum(2 * i - 1, 0), 0, 0)),
            pl.BlockSpec((B, D), lambda i: (0, 0)),
            pl.BlockSpec((B, 1), lambda i: (0, 0)),
        ],
        out_specs=pl.BlockSpec((1, 1), lambda i: (0, 0)),
        out_shape=jax.ShapeDtypeStruct((1, 1), jnp.float32),
        scratch_shapes=[
            pltpu.VMEM((B, 1), jnp.float32),
            pltpu.VMEM((B, 1), jnp.float32),
            pltpu.VMEM((B, TN), jnp.float32),
        ],
        compiler_params=pltpu.CompilerParams(
            dimension_semantics=("arbitrary",)),
    )(c, w_proj, b2d, b2d, wpt, bt2d)


def kernel(seq_index, item_indicies, target_index, W_seq, W_item, W_proj,
           b_proj):
    seq_index = seq_index.astype(jnp.int32)
    item_indicies = item_indicies.astype(jnp.int32)
    target_index = target_index.astype(jnp.int32)
    c, wpt, bt = _sc_pool(seq_index, item_indicies, target_index,
                          W_seq, W_item, W_proj, b_proj)
    loss = _tc_proj(c, W_proj,
                    (b_proj * LOG2E).reshape(NT, 1, TN), wpt,
                    bt.reshape(B, 1))
    return loss[0, 0]


# speedup vs baseline: 1.5585x; 1.5228x over previous
import functools
import math

import jax
import jax.numpy as jnp
from jax import lax
from jax.experimental import pallas as pl
from jax.experimental.pallas import tpu as pltpu
from jax.experimental.pallas import tpu_sc as plsc

NUM_ITEM = 100000
D = 128
B = 1024
WIN = 50

NC = 2
NS = 16
NW = NC * NS
BPW = B // NW
LANES = 16

TN = 5000
NT = NUM_ITEM // TN
LOG2E = math.log2(math.e)

_LOG_S2 = math.log(NUM_ITEM + 1.0 + (math.e - 2.0) / 2.0)


def _sc_pool_body(seq_idx_hbm, item_idx_hbm, tgt_idx_hbm,
                  w_seq_hbm, w_item_hbm, w_proj_hbm, b16_hbm,
                  c_hbm, wpt_hbm, bt_hbm,
                  seq_idx_v, item_idx_v, tgt_idx_v,
                  seq_rows_v, item_rows_v, wpt_rows_v, c_v, bt_v,
                  sem):
    wid = lax.axis_index("s") * NC + lax.axis_index("c")
    base = wid * BPW

    pltpu.sync_copy(seq_idx_hbm.at[pl.ds(base, BPW)], seq_idx_v)
    pltpu.sync_copy(item_idx_hbm.at[pl.ds(base, BPW)], item_idx_v)
    pltpu.sync_copy(tgt_idx_hbm.at[pl.ds(base, BPW)], tgt_idx_v)

    cp1 = pltpu.async_copy(w_seq_hbm.at[seq_idx_v], seq_rows_v, sem)
    cp2 = pltpu.async_copy(w_proj_hbm.at[tgt_idx_v], wpt_rows_v, sem)
    cp3 = pltpu.async_copy(b16_hbm.at[tgt_idx_v], bt_v, sem)
    cp1.wait()
    cp2.wait()
    cp3.wait()

    scale = LOG2E / (WIN + 1.0)
    for p in range(3):
        pltpu.async_copy(w_item_hbm.at[item_idx_v.at[p]],
                         item_rows_v.at[p], sem)

    def per_row(i, carry):
        par = lax.rem(i, 4)
        buf = item_rows_v.at[par]
        pltpu.make_async_copy(w_item_hbm.at[item_idx_v.at[i]], buf,
                              sem).wait()

        @pl.when(i + 3 < BPW)
        def _prefetch():
            pltpu.async_copy(w_item_hbm.at[item_idx_v.at[i + 3]],
                             item_rows_v.at[lax.rem(i + 3, 4)], sem)

        nch = D // LANES
        sls = [pl.ds(ch * LANES, LANES) for ch in range(nch)]

        def add_rows(g, accs):
            r = g * 10
            for k in range(10):
                accs = tuple(accs[ch] + buf[r + k, sls[ch]]
                             for ch in range(nch))
            return accs

        accs = lax.fori_loop(
            0, WIN // 10, add_rows,
            tuple(seq_rows_v[i, sls[ch]] for ch in range(nch)))
        for ch in range(nch):
            c_v[i, sls[ch]] = accs[ch] * scale
        return carry

    lax.fori_loop(0, BPW, per_row, 0)

    pltpu.sync_copy(c_v, c_hbm.at[pl.ds(base, BPW)])
    pltpu.sync_copy(wpt_rows_v, wpt_hbm.at[pl.ds(base, BPW)])
    pltpu.sync_copy(bt_v, bt_hbm.at[pl.ds(base, BPW)])


def _sc_pool(seq_idx, item_idx, tgt_idx, w_seq, w_item, w_proj, b16):
    mesh = plsc.VectorSubcoreMesh(core_axis_name="c", subcore_axis_name="s")
    f = pl.kernel(
        _sc_pool_body,
        out_type=[
            jax.ShapeDtypeStruct((B, D), jnp.float32),
            jax.ShapeDtypeStruct((B, D), jnp.float32),
            jax.ShapeDtypeStruct((B,), jnp.float32),
        ],
        mesh=mesh,
        scratch_types=[
            pltpu.VMEM((BPW,), jnp.int32),
            pltpu.VMEM((BPW, WIN), jnp.int32),
            pltpu.VMEM((BPW,), jnp.int32),
            pltpu.VMEM((BPW, D), jnp.float32),
            pltpu.VMEM((4, WIN, D), jnp.float32),
            pltpu.VMEM((BPW, D), jnp.float32),
            pltpu.VMEM((BPW, D), jnp.float32),
            pltpu.VMEM((BPW,), jnp.float32),
            pltpu.SemaphoreType.DMA,
        ],
    )
    return f(seq_idx, item_idx, tgt_idx, w_seq, w_item, w_proj, b16)


def _tc_proj_body(c_ref, w_ref, b_ref, wpt_ref, bt_ref, loss_ref,
                  m_ref, z_ref):
    j = pl.program_id(0)

    @pl.when(j == 0)
    def _init():
        m_ref[...] = jnp.full((B, 1), -jnp.inf, jnp.float32)
        z_ref[...] = jnp.zeros((B, 1), jnp.float32)

    cb = c_ref[...].astype(jnp.bfloat16)
    dn = (((1,), (1,)), ((), ()))
    NS_ = 2
    H = TN // NS_
    b = b_ref[0]
    ds_ = [lax.dot_general(cb,
                           w_ref[pl.ds(k * H, H), :].astype(jnp.bfloat16),
                           dn, preferred_element_type=jnp.float32)
           for k in range(NS_)]
    bs_ = [b[:, k * H:(k + 1) * H] for k in range(NS_)]
    m_old = m_ref[...]
    s_u = sum(jnp.sum(jnp.exp2(ds_[k] + bs_[k]), axis=1, keepdims=True)
              for k in range(NS_))
    s = s_u * jnp.exp2(-m_old)
    trip = (jnp.any(s > 2.0 ** 100) | jnp.any(s_u < 2.0 ** -100)
            | jnp.any(jnp.isnan(s)))

    @pl.when(jnp.logical_not(trip))
    def _fast():
        z_ref[...] = z_ref[...] + s

    @pl.when(trip)
    def _rebase():
        bound = (functools.reduce(
            jnp.maximum,
            [jnp.max(ds_[k], axis=1, keepdims=True) for k in range(NS_)])
            + jnp.max(b))
        m_new = jnp.maximum(m_old, bound)
        z_ref[...] = (z_ref[...] * jnp.exp2(m_old - m_new)
                      + sum(jnp.sum(jnp.exp2((ds_[k] + bs_[k]) - m_new),
                                    axis=1, keepdims=True)
                            for k in range(NS_)))
        m_ref[...] = m_new

    @pl.when(j == NT - 1)
    def _fin():
        vt = jnp.sum(c_ref[...] * wpt_ref[...], axis=1, keepdims=True)
        vt = vt + bt_ref[...] * LOG2E
        out_t = jnp.exp2(vt - m_ref[...]) / z_ref[...]
        loss_ref[...] = jnp.reshape(
            _LOG_S2 - jnp.sum(out_t) * (1.0 / B), (1, 1))


def _tc_proj(c, w_proj, b2d, wpt, bt2d):
    return pl.pallas_call(
        _tc_proj_body,
        grid=(NT,),
        in_specs=[
            pl.BlockSpec((B, D), lambda j: (0, 0)),
            pl.BlockSpec((TN, D), lambda j: (j, 0)),
            pl.BlockSpec((1, 1, TN), lambda j: (j, 0, 0)),
            pl.BlockSpec((B, D), lambda j: (0, 0)),
            pl.BlockSpec((B, 1), lambda j: (0, 0)),
        ],
        out_specs=pl.BlockSpec((1, 1), lambda j: (0, 0)),
        out_shape=jax.ShapeDtypeStruct((1, 1), jnp.float32),
        scratch_shapes=[
            pltpu.VMEM((B, 1), jnp.float32),
            pltpu.VMEM((B, 1), jnp.float32),
        ],
        compiler_params=pltpu.CompilerParams(
            dimension_semantics=("arbitrary",)),
    )(c, w_proj, b2d, wpt, bt2d)


def kernel(seq_index, item_indicies, target_index, W_seq, W_item, W_proj,
           b_proj):
    seq_index = seq_index.astype(jnp.int32)
    item_indicies = item_indicies.astype(jnp.int32)
    target_index = target_index.astype(jnp.int32)
    c, wpt, bt = _sc_pool(seq_index, item_indicies, target_index,
                          W_seq, W_item, W_proj, b_proj)
    loss = _tc_proj(c, W_proj,
                    (b_proj * LOG2E).reshape(NT, 1, TN), wpt,
                    bt.reshape(B, 1))
    return loss[0, 0]
